# Initial kernel scaffold; baseline (speedup 1.0000x reference)
import jax
import jax.numpy as jnp
from jax.experimental import pallas as pl


def kernel(queries, keys, selection_mask, k):
    raise NotImplementedError("write your pallas kernel here")


# speedup vs baseline: 3.0494x; 3.0494x over previous
import functools

import jax
import jax.numpy as jnp
from jax import lax
from jax.experimental import pallas as pl
from jax.experimental.pallas import tpu as pltpu
from jax.experimental.pallas import tpu_sc as plsc

Q = 1024
K_POOL = 100000
D = 128
TOPK = 8
GSIZE = 128
BK = 2048
NKSTEP = 49
KPAD = NKSTEP * BK
G = KPAD // GSIZE
TPB = BK // GSIZE
BIGF = 3.0e38
IBIG = 2**30


def _dist_kernel(q_ref, k_ref, dist_ref, gids_ref, rows_ref, qn_ref,
                 rv_ref, rid_ref):
    ki = pl.program_id(0)

    @pl.when(ki == 0)
    def _():
        q = q_ref[...]
        n = jnp.sqrt(jnp.sum(q * q, axis=-1, keepdims=True))
        qn_ref[...] = q / jnp.maximum(n, 1e-12)
        rv_ref[...] = jnp.full((Q, TOPK), BIGF, jnp.float32)
        rid_ref[...] = jnp.full((Q, TOPK), IBIG, jnp.int32)

    k = k_ref[...]
    kn = k / jnp.maximum(jnp.sqrt(jnp.sum(k * k, axis=-1, keepdims=True)), 1e-12)
    sim = jax.lax.dot_general(qn_ref[...], kn, (((1,), (1,)), ((), ())),
                              preferred_element_type=jnp.float32)
    dist = 2.0 - 2.0 * sim
    gk = ki * BK + lax.broadcasted_iota(jnp.int32, (Q, BK), 1)
    dist = jnp.where(gk < K_POOL, dist, BIGF)
    dist_ref[...] = dist
    gmin = jnp.min(dist.reshape(Q, TPB, GSIZE), axis=-1)

    v = jnp.concatenate([rv_ref[...], gmin], axis=-1)
    gi = jnp.concatenate(
        [rid_ref[...],
         ki * TPB + lax.broadcasted_iota(jnp.int32, (Q, TPB), 1)], axis=-1)
    vcols, icols = [], []
    for _ in range(TOPK):
        m = jnp.min(v, axis=-1, keepdims=True)
        gid = jnp.min(jnp.where(v == m, gi, IBIG), axis=-1, keepdims=True)
        vcols.append(m)
        icols.append(gid)
        v = jnp.where(gi == gid, BIGF, v)
    rv_ref[...] = jnp.concatenate(vcols, axis=-1)
    rid_ref[...] = jnp.concatenate(icols, axis=-1)

    @pl.when(ki == NKSTEP - 1)
    def _():
        gids = rid_ref[...]
        gids_ref[...] = gids
        qrow = lax.broadcasted_iota(jnp.int32, (Q, TOPK), 0)
        rows_ref[...] = qrow * G + gids


def _compute_dist(queries, keys):
    return pl.pallas_call(
        _dist_kernel,
        grid=(NKSTEP,),
        in_specs=[
            pl.BlockSpec((Q, D), lambda ki: (0, 0)),
            pl.BlockSpec((BK, D), lambda ki: (ki, 0)),
        ],
        out_specs=[
            pl.BlockSpec((Q, BK), lambda ki: (0, ki)),
            pl.BlockSpec((Q, TOPK), lambda ki: (0, 0)),
            pl.BlockSpec((Q, TOPK), lambda ki: (0, 0)),
        ],
        out_shape=[
            jax.ShapeDtypeStruct((Q, KPAD), jnp.float32),
            jax.ShapeDtypeStruct((Q, TOPK), jnp.int32),
            jax.ShapeDtypeStruct((Q, TOPK), jnp.int32),
        ],
        scratch_shapes=[
            pltpu.VMEM((Q, D), jnp.float32),
            pltpu.VMEM((Q, TOPK), jnp.float32),
            pltpu.VMEM((Q, TOPK), jnp.int32),
        ],
    )(queries, keys)


def _sc_gather(table, idx, nrows, width):
    info = plsc.get_sparse_core_info()
    nw = info.num_cores * info.num_subcores
    b_per_w = nrows // nw
    mesh = plsc.VectorSubcoreMesh(core_axis_name="c", subcore_axis_name="s")

    @functools.partial(
        pl.kernel, mesh=mesh,
        out_type=jax.ShapeDtypeStruct((nrows, width), jnp.float32),
        scratch_types=[
            pltpu.VMEM((b_per_w,), jnp.int32),
            pltpu.VMEM((b_per_w, width), jnp.float32),
            pltpu.SemaphoreType.DMA,
        ],
    )
    def k(table_hbm, idx_hbm, out_hbm, idx_v, rows_v, sem):
        wid = lax.axis_index("s") * info.num_cores + lax.axis_index("c")
        base = wid * b_per_w
        pltpu.sync_copy(idx_hbm.at[pl.ds(base, b_per_w)], idx_v)
        pltpu.async_copy(table_hbm.at[idx_v], rows_v, sem).wait()
        pltpu.sync_copy(rows_v, out_hbm.at[pl.ds(base, b_per_w)])

    return k(table, idx)


def _final_kernel(cand_ref, gids_ref, mask_ref, ids_ref):
    NC = TOPK * GSIZE
    d = cand_ref[...]
    c = lax.broadcasted_iota(jnp.int32, (Q, NC), 1)
    j = c // GSIZE
    pos = c - j * GSIZE
    gidx = jnp.zeros((Q, NC), jnp.int32)
    for t in range(TOPK):
        gidx = gidx + jnp.where(j == t, gids_ref[:, t][:, None], 0)
    gidx = gidx * GSIZE + pos
    mask = mask_ref[...]
    ids = jnp.zeros((Q, 1), jnp.int32)
    for i in range(TOPK):
        m = jnp.min(d, axis=-1, keepdims=True)
        gid = jnp.min(jnp.where(d == m, gidx, IBIG), axis=-1, keepdims=True)
        ids = ids + jnp.where(mask == i, gid, 0)
        d = jnp.where(gidx == gid, BIGF, d)
    ids_ref[...] = ids


def _final_topk(cand, gids, mask2d):
    return pl.pallas_call(
        _final_kernel,
        out_shape=jax.ShapeDtypeStruct((Q, 1), jnp.int32),
    )(cand, gids, mask2d)


def kernel(queries, keys, selection_mask, k):
    dist, gids, rows = _compute_dist(queries, keys)
    cand = _sc_gather(dist.reshape(Q * G, GSIZE), rows.reshape(-1),
                      Q * TOPK, GSIZE)
    mask2d = selection_mask.astype(jnp.int32).reshape(Q, 1)
    ids = _final_topk(cand.reshape(Q, TOPK * GSIZE), gids, mask2d)[:, 0]
    neighbors = _sc_gather(keys, ids, Q, D)
    return neighbors, ids


# speedup vs baseline: 6.0918x; 1.9977x over previous
import functools

import jax
import jax.numpy as jnp
from jax import lax
from jax.experimental import pallas as pl
from jax.experimental.pallas import tpu as pltpu
from jax.experimental.pallas import tpu_sc as plsc

Q = 1024
K_POOL = 100000
D = 128
TOPK = 8
GSIZE = 128
BK = 4096
NKSTEP = 25
KPAD = NKSTEP * BK
G = KPAD // GSIZE
TPB = BK // GSIZE
BIGF = 3.0e38
IBIG = 2**30


def _dist_kernel(q_ref, k_ref, dist_ref, gids_ref, rows_ref, qn_ref,
                 rv_ref, rid_ref):
    ki = pl.program_id(0)

    @pl.when(ki == 0)
    def _():
        q = q_ref[...]
        n = jnp.sqrt(jnp.sum(q * q, axis=-1, keepdims=True))
        qn_ref[...] = q / jnp.maximum(n, 1e-12)
        rv_ref[...] = jnp.full((Q, TOPK), BIGF, jnp.float32)
        rid_ref[...] = jnp.full((Q, TOPK), IBIG, jnp.int32)

    k = k_ref[...]
    kn = k / jnp.maximum(jnp.sqrt(jnp.sum(k * k, axis=-1, keepdims=True)), 1e-12)
    sim = jax.lax.dot_general(qn_ref[...], kn, (((1,), (1,)), ((), ())),
                              preferred_element_type=jnp.float32)
    dist = 2.0 - 2.0 * sim
    gk = ki * BK + lax.broadcasted_iota(jnp.int32, (Q, BK), 1)
    dist = jnp.where(gk < K_POOL, dist, BIGF)
    dist3 = dist.reshape(Q, TPB, GSIZE)
    dist_ref[...] = dist3
    gmin = jnp.min(dist3, axis=-1)

    v = jnp.concatenate([rv_ref[...], gmin], axis=-1)
    gi = jnp.concatenate(
        [rid_ref[...],
         ki * TPB + lax.broadcasted_iota(jnp.int32, (Q, TPB), 1)], axis=-1)
    vcols, icols = [], []
    for _ in range(TOPK):
        m = jnp.min(v, axis=-1, keepdims=True)
        gid = jnp.min(jnp.where(v == m, gi, IBIG), axis=-1, keepdims=True)
        vcols.append(m)
        icols.append(gid)
        v = jnp.where(gi == gid, BIGF, v)
    rv_ref[...] = jnp.concatenate(vcols, axis=-1)
    rid_ref[...] = jnp.concatenate(icols, axis=-1)

    @pl.when(ki == NKSTEP - 1)
    def _():
        gids = rid_ref[...]
        gids_ref[...] = gids
        qrow = lax.broadcasted_iota(jnp.int32, (Q, TOPK), 0)
        rows_ref[...] = qrow * G + gids


def _compute_dist(queries, keys):
    return pl.pallas_call(
        _dist_kernel,
        grid=(NKSTEP,),
        in_specs=[
            pl.BlockSpec((Q, D), lambda ki: (0, 0)),
            pl.BlockSpec((BK, D), lambda ki: (ki, 0)),
        ],
        out_specs=[
            pl.BlockSpec((Q, TPB, GSIZE), lambda ki: (0, ki, 0)),
            pl.BlockSpec((Q, TOPK), lambda ki: (0, 0)),
            pl.BlockSpec((Q, TOPK), lambda ki: (0, 0)),
        ],
        out_shape=[
            jax.ShapeDtypeStruct((Q, G, GSIZE), jnp.float32),
            jax.ShapeDtypeStruct((Q, TOPK), jnp.int32),
            jax.ShapeDtypeStruct((Q, TOPK), jnp.int32),
        ],
        scratch_shapes=[
            pltpu.VMEM((Q, D), jnp.float32),
            pltpu.VMEM((Q, TOPK), jnp.float32),
            pltpu.VMEM((Q, TOPK), jnp.int32),
        ],
    )(queries, keys)


def _sc_gather(table, idx, nrows, width):
    info = plsc.get_sparse_core_info()
    nw = info.num_cores * info.num_subcores
    b_per_w = nrows // nw
    mesh = plsc.VectorSubcoreMesh(core_axis_name="c", subcore_axis_name="s")

    @functools.partial(
        pl.kernel, mesh=mesh,
        out_type=jax.ShapeDtypeStruct((nrows, width), jnp.float32),
        scratch_types=[
            pltpu.VMEM((b_per_w,), jnp.int32),
            pltpu.VMEM((b_per_w, width), jnp.float32),
            pltpu.SemaphoreType.DMA,
        ],
    )
    def k(table_hbm, idx_hbm, out_hbm, idx_v, rows_v, sem):
        wid = lax.axis_index("s") * info.num_cores + lax.axis_index("c")
        base = wid * b_per_w
        pltpu.sync_copy(idx_hbm.at[pl.ds(base, b_per_w)], idx_v)
        pltpu.async_copy(table_hbm.at[idx_v], rows_v, sem).wait()
        pltpu.sync_copy(rows_v, out_hbm.at[pl.ds(base, b_per_w)])

    return k(table, idx)


def _final_kernel(cand_ref, gids_ref, mask_ref, ids_ref):
    NC = TOPK * GSIZE
    d = cand_ref[...]
    c = lax.broadcasted_iota(jnp.int32, (Q, NC), 1)
    j = c // GSIZE
    pos = c - j * GSIZE
    gidx = jnp.zeros((Q, NC), jnp.int32)
    for t in range(TOPK):
        gidx = gidx + jnp.where(j == t, gids_ref[:, t][:, None], 0)
    gidx = gidx * GSIZE + pos
    mask = mask_ref[...]
    ids = jnp.zeros((Q, 1), jnp.int32)
    for i in range(TOPK):
        m = jnp.min(d, axis=-1, keepdims=True)
        gid = jnp.min(jnp.where(d == m, gidx, IBIG), axis=-1, keepdims=True)
        ids = ids + jnp.where(mask == i, gid, 0)
        d = jnp.where(gidx == gid, BIGF, d)
    ids_ref[...] = ids


def _final_topk(cand, gids, mask2d):
    return pl.pallas_call(
        _final_kernel,
        out_shape=jax.ShapeDtypeStruct((Q, 1), jnp.int32),
    )(cand, gids, mask2d)


def kernel(queries, keys, selection_mask, k):
    dist3, gids, rows = _compute_dist(queries, keys)
    cand = _sc_gather(dist3.reshape(Q * G, GSIZE), rows.reshape(-1),
                      Q * TOPK, GSIZE)
    mask2d = selection_mask.astype(jnp.int32).reshape(Q, 1)
    ids = _final_topk(cand.reshape(Q, TOPK * GSIZE), gids, mask2d)[:, 0]
    neighbors = _sc_gather(keys, ids, Q, D)
    return neighbors, ids


# speedup vs baseline: 6.9602x; 1.1425x over previous
import functools

import jax
import jax.numpy as jnp
from jax import lax
from jax.experimental import pallas as pl
from jax.experimental.pallas import tpu as pltpu
from jax.experimental.pallas import tpu_sc as plsc

Q = 1024
K_POOL = 100000
D = 128
TOPK = 8
GSIZE = 128
BK = 4096
NKSTEP = 25
KPAD = NKSTEP * BK
G = KPAD // GSIZE
TPB = BK // GSIZE
BIGF = 3.0e38
IBIG = 2**30


def _dist_kernel(q_ref, k_ref, pad_ref, dist_ref, gids_ref, rows_ref, qn_ref,
                 rv_ref, rid_ref):
    ki = pl.program_id(0)

    @pl.when(ki == 0)
    def _():
        q = q_ref[...]
        n = jnp.sqrt(jnp.sum(q * q, axis=-1, keepdims=True))
        qn_ref[...] = q / jnp.maximum(n, 1e-12)
        rv_ref[...] = jnp.full((Q, TOPK), BIGF, jnp.float32)
        rid_ref[...] = jnp.full((Q, TOPK), float(IBIG), jnp.float32)

    k = k_ref[...]
    k = jnp.where(jnp.abs(k) < 1e30, k, 0.0)
    kn = k / jnp.maximum(jnp.sqrt(jnp.sum(k * k, axis=-1, keepdims=True)), 1e-12)
    sim = jax.lax.dot_general(qn_ref[...], kn, (((1,), (1,)), ((), ())),
                              preferred_element_type=jnp.float32)
    dist = (2.0 - 2.0 * sim) + pad_ref[0:1, :]
    dist3 = dist.reshape(Q, TPB, GSIZE)
    dist_ref[...] = dist3
    gmin = jnp.min(dist3, axis=-1)

    v = jnp.concatenate([rv_ref[...], gmin], axis=-1)
    gi = jnp.concatenate(
        [rid_ref[...],
         (ki * TPB + lax.broadcasted_iota(jnp.int32, (Q, TPB), 1)
          ).astype(jnp.float32)], axis=-1)
    vcols, icols = [], []
    for _ in range(TOPK):
        m = jnp.min(v, axis=-1, keepdims=True)
        gid = jnp.min(jnp.where(v == m, gi, float(IBIG)), axis=-1,
                      keepdims=True)
        vcols.append(m)
        icols.append(gid)
        v = jnp.where(gi == gid, BIGF, v)
    rv_ref[...] = jnp.concatenate(vcols, axis=-1)
    rid_ref[...] = jnp.concatenate(icols, axis=-1)

    @pl.when(ki == NKSTEP - 1)
    def _():
        gids = rid_ref[...].astype(jnp.int32)
        gids_ref[...] = gids
        qrow = lax.broadcasted_iota(jnp.int32, (Q, TOPK), 0)
        rows_ref[...] = qrow * G + gids


def _compute_dist(queries, keys):
    return pl.pallas_call(
        _dist_kernel,
        grid=(NKSTEP,),
        in_specs=[
            pl.BlockSpec((Q, D), lambda ki: (0, 0)),
            pl.BlockSpec((BK, D), lambda ki: (ki, 0)),
            pl.BlockSpec((8, BK), lambda ki: (0, ki)),
        ],
        out_specs=[
            pl.BlockSpec((Q, TPB, GSIZE), lambda ki: (0, ki, 0)),
            pl.BlockSpec((Q, TOPK), lambda ki: (0, 0)),
            pl.BlockSpec((Q, TOPK), lambda ki: (0, 0)),
        ],
        out_shape=[
            jax.ShapeDtypeStruct((Q, G, GSIZE), jnp.float32),
            jax.ShapeDtypeStruct((Q, TOPK), jnp.int32),
            jax.ShapeDtypeStruct((Q, TOPK), jnp.int32),
        ],
        scratch_shapes=[
            pltpu.VMEM((Q, D), jnp.float32),
            pltpu.VMEM((Q, TOPK), jnp.float32),
            pltpu.VMEM((Q, TOPK), jnp.float32),
        ],
    )(queries, keys, _pad_row())


def _pad_row():
    row = jnp.where(jnp.arange(KPAD) < K_POOL, 0.0, BIGF).astype(jnp.float32)
    return jnp.broadcast_to(row[None, :], (8, KPAD))


def _sc_gather(table, idx, nrows, width):
    info = plsc.get_sparse_core_info()
    nw = info.num_cores * info.num_subcores
    b_per_w = nrows // nw
    mesh = plsc.VectorSubcoreMesh(core_axis_name="c", subcore_axis_name="s")

    @functools.partial(
        pl.kernel, mesh=mesh,
        out_type=jax.ShapeDtypeStruct((nrows, width), jnp.float32),
        scratch_types=[
            pltpu.VMEM((b_per_w,), jnp.int32),
            pltpu.VMEM((b_per_w, width), jnp.float32),
            pltpu.SemaphoreType.DMA,
        ],
    )
    def k(table_hbm, idx_hbm, out_hbm, idx_v, rows_v, sem):
        wid = lax.axis_index("s") * info.num_cores + lax.axis_index("c")
        base = wid * b_per_w
        pltpu.sync_copy(idx_hbm.at[pl.ds(base, b_per_w)], idx_v)
        pltpu.async_copy(table_hbm.at[idx_v], rows_v, sem).wait()
        pltpu.sync_copy(rows_v, out_hbm.at[pl.ds(base, b_per_w)])

    return k(table, idx)


def _final_kernel(cand_ref, gids_ref, mask_ref, ids_ref):
    NC = TOPK * GSIZE
    d = cand_ref[...]
    c = lax.broadcasted_iota(jnp.int32, (Q, NC), 1)
    j = c // GSIZE
    pos = c - j * GSIZE
    gidx = jnp.zeros((Q, NC), jnp.int32)
    for t in range(TOPK):
        gidx = gidx + jnp.where(j == t, gids_ref[:, t][:, None], 0)
    gidx = gidx * GSIZE + pos
    mask = mask_ref[...]
    ids = jnp.zeros((Q, 1), jnp.int32)
    for i in range(TOPK):
        m = jnp.min(d, axis=-1, keepdims=True)
        gid = jnp.min(jnp.where(d == m, gidx, IBIG), axis=-1, keepdims=True)
        ids = ids + jnp.where(mask == i, gid, 0)
        d = jnp.where(gidx == gid, BIGF, d)
    ids_ref[...] = ids


def _final_topk(cand, gids, mask2d):
    return pl.pallas_call(
        _final_kernel,
        out_shape=jax.ShapeDtypeStruct((Q, 1), jnp.int32),
    )(cand, gids, mask2d)


def kernel(queries, keys, selection_mask, k):
    dist3, gids, rows = _compute_dist(queries, keys)
    cand = _sc_gather(dist3.reshape(Q * G, GSIZE), rows.reshape(-1),
                      Q * TOPK, GSIZE)
    mask2d = selection_mask.astype(jnp.int32).reshape(Q, 1)
    ids = _final_topk(cand.reshape(Q, TOPK * GSIZE), gids, mask2d)[:, 0]
    neighbors = _sc_gather(keys, ids, Q, D)
    return neighbors, ids
